# baseline (speedup 1.0000x reference)
import functools

import jax
import jax.numpy as jnp
from jax import lax
from jax.experimental import pallas as pl
from jax.experimental.pallas import tpu as pltpu
from jax.experimental.pallas import tpu_sc as plsc

L = 16
LANE = 128


def _build(E, N, T, R, G, NO, NC, NS):
    NW = NC * NS
    CE = 1024
    NCH = E // CE
    ncw = NCH // NW
    rem = NCH - ncw * NW
    RT = -(-R // 8)
    RH = R - 8
    TPC = CE // 128
    CA = 3200
    NPAD = CA * NW

    mesh = plsc.VectorSubcoreMesh(core_axis_name="c", subcore_axis_name="s")

    @functools.partial(
        pl.kernel,
        mesh=mesh,
        out_type=(
            jax.ShapeDtypeStruct((RT * E // 128, 8 * 128), jnp.float32),
            jax.ShapeDtypeStruct((NPAD * NO,), jnp.float32),
        ),
        scratch_types=[
            pltpu.VMEM((T * G * R,), jnp.int32),
            pltpu.VMEM((G,), jnp.float32),
            pltpu.VMEM((L,), jnp.float32),
            pltpu.VMEM((CE,), jnp.float32),
            pltpu.VMEM((CE,), jnp.float32),
            pltpu.VMEM((CE,), jnp.int32),
            pltpu.VMEM((CE,), jnp.int32),
            pltpu.VMEM((TPC, 8 * 128), jnp.float32),
            pltpu.VMEM((TPC, 8 * 128), jnp.float32),
            pltpu.VMEM((TPC, RH * 128), jnp.float32),
            pltpu.VMEM((TPC, RH * 128), jnp.float32),
            pltpu.VMEM((CA,), jnp.int32),
            pltpu.VMEM((CA * NO,), jnp.float32),
            pltpu.SemaphoreType.DMA,
            pltpu.SemaphoreType.DMA,
            pltpu.SemaphoreType.DMA,
            pltpu.SemaphoreType.DMA,
        ],
        compiler_params=pltpu.CompilerParams(
            needs_layout_passes=False, disable_bounds_checks=True),
    )
    def sc_kernel(rij_hbm, et_hbm, at_hbm, grid_hbm, tab_hbm, ons_hbm,
                  oute_hbm, outn_hbm,
                  tab_v, grid_v, ons_v, rij_v0, rij_v1, et_v0, et_v1,
                  oute0_v0, oute0_v1, oute1_v0, oute1_v1, at_v, outn_v,
                  sin0, sin1, sout0, sout1):
        wid = lax.axis_index("s") * NC + lax.axis_index("c")
        lo = ncw * wid + jnp.minimum(wid, rem)
        cnt = ncw + (wid < rem).astype(jnp.int32)

        rij_vs = (rij_v0, rij_v1)
        et_vs = (et_v0, et_v1)
        oute0_vs = (oute0_v0, oute0_v1)
        oute1_vs = (oute1_v0, oute1_v1)
        sins = (sin0, sin1)
        souts = (sout0, sout1)

        pltpu.sync_copy(tab_hbm, tab_v)
        pltpu.sync_copy(grid_hbm, grid_v)
        pltpu.sync_copy(ons_hbm, ons_v)

        d0 = grid_v[pl.ds(0, L)][0]
        dlast = grid_v[pl.ds(G - L, L)][L - 1]
        ii = lax.iota(jnp.int32, L)
        inv_dx = (jnp.zeros((L,), jnp.float32) + jnp.float32(G - 1)) / (dlast - d0)

        def start_in(c, b):
            eb = c * CE
            pltpu.async_copy(rij_hbm.at[pl.ds(eb, CE)], rij_vs[b], sins[b])
            pltpu.async_copy(et_hbm.at[pl.ds(eb, CE)], et_vs[b], sins[b])

        def wait_in(b):
            pltpu.make_async_copy(rij_hbm.at[pl.ds(0, CE)], rij_vs[b], sins[b]).wait()
            pltpu.make_async_copy(et_hbm.at[pl.ds(0, CE)], et_vs[b], sins[b]).wait()

        def start_out(c, b):
            pltpu.async_copy(
                oute0_vs[b], oute_hbm.at[pl.ds(c * TPC, TPC), pl.ds(0, 8 * 128)],
                souts[b])
            pltpu.async_copy(
                oute1_vs[b],
                oute_hbm.at[pl.ds(E // 128 + c * TPC, TPC), pl.ds(0, RH * 128)],
                souts[b])

        def wait_out(b):
            pltpu.make_async_copy(
                oute0_vs[b], oute_hbm.at[pl.ds(0, TPC), pl.ds(0, 8 * 128)],
                souts[b]).wait()
            pltpu.make_async_copy(
                oute1_vs[b],
                oute_hbm.at[pl.ds(0, TPC), pl.ds(0, RH * 128)], souts[b]).wait()

        def make_edge_group(rij_v, et_v, oute0_v, oute1_v):
            def edge_group(s):
                r = rij_v[pl.ds(s, L)]
                t = et_v[pl.ds(s, L)]
                rf = (r - d0) * inv_dx
                i0 = rf.astype(jnp.int32)
                i0 = jnp.clip(i0, 0, G - 2)
                w = rf - i0.astype(jnp.float32)
                base = (t * G + i0) * R
                blk = s // LANE
                off = lax.rem(s, LANE)
                hi = jnp.full((L,), -65536, jnp.int32)
                for rr in range(R):
                    yp = plsc.load_gather(tab_v, [base + rr])
                    y0 = plsc.bitcast(yp << 16, jnp.float32)
                    y1 = plsc.bitcast(yp & hi, jnp.float32)
                    o = y0 + w * (y1 - y0)
                    if rr < 8:
                        oute0_v[blk, pl.ds(rr * LANE + off, L)] = o
                    else:
                        oute1_v[blk, pl.ds((rr - 8) * LANE + off, L)] = o
            return edge_group

        groups = tuple(
            make_edge_group(rij_vs[b], et_vs[b], oute0_vs[b], oute1_vs[b])
            for b in range(2))

        start_in(lo, 0)
        start_in(lo + 1, 1)

        def edge_pair(k, _):
            for b in range(2):
                crel = 2 * k + b

                @pl.when(crel < cnt)
                def _():
                    wait_in(b)

                    @pl.when(crel >= 2)
                    def _():
                        wait_out(b)

                    plsc.parallel_loop(0, CE, step=L, unroll=4)(groups[b])
                    start_out(lo + crel, b)

                    @pl.when(crel + 2 < cnt)
                    def _():
                        start_in(lo + crel + 2, b)
            return 0

        lax.fori_loop(0, (ncw + 2) // 2, edge_pair, 0)

        abase = wid * CA
        pltpu.sync_copy(at_hbm.at[pl.ds(abase, CA)], at_v)

        @plsc.parallel_loop(0, CA, step=L, unroll=4)
        def atom_group(s):
            at = at_v[pl.ds(s, L)]
            ab = at * NO
            ob = (s // LANE) * (NO * LANE) + lax.rem(s, LANE)
            for oo in range(NO):
                v = plsc.load_gather(ons_v, [ab + oo])
                outn_v[pl.ds(ob + oo * LANE, L)] = v

        pltpu.sync_copy(outn_v, outn_hbm.at[pl.ds(abase * NO, CA * NO)])

        wait_out(0)
        wait_out(1)

    return sc_kernel, NPAD, RT


def kernel(rij, edge_type, atom_type, distance_param, hopping_param, onsite_param):
    E = rij.shape[0]
    N = atom_type.shape[0]
    T, R, G = hopping_param.shape
    NO = onsite_param.shape[1]

    info = plsc.get_sparse_core_info()
    NC, NS = info.num_cores, info.num_subcores

    sc_kernel, NPAD, RT = _build(E, N, T, R, G, NO, NC, NS)

    tgr = jnp.transpose(hopping_param, (0, 2, 1))
    y0b = tgr.astype(jnp.bfloat16)
    y1b = jnp.concatenate([tgr[:, 1:], tgr[:, -1:]], axis=1).astype(jnp.bfloat16)
    lo16 = (jax.lax.bitcast_convert_type(y0b, jnp.uint16)).astype(jnp.uint32)
    hi16 = (jax.lax.bitcast_convert_type(y1b, jnp.uint16)).astype(jnp.uint32)
    tab = jax.lax.bitcast_convert_type(lo16 | (hi16 << 16), jnp.int32).reshape(-1)
    A = onsite_param.shape[0]
    ons = jnp.zeros((L,), jnp.float32).at[: A * NO].set(
        onsite_param[:, :, 0].reshape(-1).astype(jnp.float32))
    at_pad = jnp.zeros((NPAD,), jnp.int32).at[:N].set(atom_type.astype(jnp.int32))

    edge_flat, node_flat = sc_kernel(
        rij, edge_type.astype(jnp.int32), at_pad,
        distance_param, tab, ons)

    ef = edge_flat.reshape(RT, E // 128, 8, 128)
    ef = jnp.transpose(ef, (1, 3, 0, 2)).reshape(E, RT * 8)[:, :R]
    nf = node_flat.reshape(NPAD // 128, NO, 128)
    nf = jnp.transpose(nf, (0, 2, 1)).reshape(NPAD, NO)[:N]
    return ef, nf


# speedup vs baseline: 2.7730x; 2.7730x over previous
import functools

import jax
import jax.numpy as jnp
from jax import lax
from jax.experimental import pallas as pl
from jax.experimental.pallas import tpu as pltpu
from jax.experimental.pallas import tpu_sc as plsc

L = 16
LANE = 128


def _build(E, N, T, R, G, NO, NC, NS):
    NW = NC * NS
    CE = 2560
    NCH = E // CE
    ncw = NCH // NW
    rem = NCH - ncw * NW
    RT = -(-R // 8)
    CA = 3200
    NPAD = CA * NW

    mesh = plsc.VectorSubcoreMesh(core_axis_name="c", subcore_axis_name="s")

    @functools.partial(
        pl.kernel,
        mesh=mesh,
        out_type=(
            jax.ShapeDtypeStruct((E * 8 * RT,), jnp.float32),
            jax.ShapeDtypeStruct((NPAD * NO,), jnp.float32),
        ),
        scratch_types=[
            pltpu.VMEM((T * G * R,), jnp.int32),
            pltpu.VMEM((G,), jnp.float32),
            pltpu.VMEM((L,), jnp.float32),
            pltpu.VMEM((CE,), jnp.float32),
            pltpu.VMEM((CE,), jnp.float32),
            pltpu.VMEM((CE,), jnp.int32),
            pltpu.VMEM((CE,), jnp.int32),
            pltpu.VMEM((CE * 8 * RT,), jnp.float32),
            pltpu.VMEM((CE * 8 * RT,), jnp.float32),
            pltpu.VMEM((CA,), jnp.int32),
            pltpu.VMEM((CA * NO,), jnp.float32),
            pltpu.SemaphoreType.DMA,
            pltpu.SemaphoreType.DMA,
            pltpu.SemaphoreType.DMA,
            pltpu.SemaphoreType.DMA,
        ],
        compiler_params=pltpu.CompilerParams(
            needs_layout_passes=False, disable_bounds_checks=True),
    )
    def sc_kernel(rij_hbm, et_hbm, at_hbm, grid_hbm, tab_hbm, ons_hbm,
                  oute_hbm, outn_hbm,
                  tab_v, grid_v, ons_v, rij_v0, rij_v1, et_v0, et_v1,
                  oute_v0, oute_v1, at_v, outn_v,
                  sin0, sin1, sout0, sout1):
        wid = lax.axis_index("s") * NC + lax.axis_index("c")
        lo = ncw * wid + jnp.minimum(wid, rem)
        cnt = ncw + (wid < rem).astype(jnp.int32)

        rij_vs = (rij_v0, rij_v1)
        et_vs = (et_v0, et_v1)
        oute_vs = (oute_v0, oute_v1)
        sins = (sin0, sin1)
        souts = (sout0, sout1)

        pltpu.sync_copy(tab_hbm, tab_v)
        pltpu.sync_copy(grid_hbm, grid_v)
        pltpu.sync_copy(ons_hbm, ons_v)

        d0 = grid_v[pl.ds(0, L)][0]
        dlast = grid_v[pl.ds(G - L, L)][L - 1]
        ii = lax.iota(jnp.int32, L)
        inv_dx = (jnp.zeros((L,), jnp.float32) + jnp.float32(G - 1)) / (dlast - d0)

        def start_in(c, b):
            eb = c * CE
            pltpu.async_copy(rij_hbm.at[pl.ds(eb, CE)], rij_vs[b], sins[b])
            pltpu.async_copy(et_hbm.at[pl.ds(eb, CE)], et_vs[b], sins[b])

        def wait_in(b):
            pltpu.make_async_copy(rij_hbm.at[pl.ds(0, CE)], rij_vs[b], sins[b]).wait()
            pltpu.make_async_copy(et_hbm.at[pl.ds(0, CE)], et_vs[b], sins[b]).wait()

        def start_out(c, b):
            eb8 = c * CE * 8
            for rt in range(RT):
                pltpu.async_copy(
                    oute_vs[b].at[pl.ds(rt * CE * 8, CE * 8)],
                    oute_hbm.at[pl.ds(rt * E * 8 + eb8, CE * 8)], souts[b])

        def wait_out(b):
            for rt in range(RT):
                pltpu.make_async_copy(
                    oute_vs[b].at[pl.ds(rt * CE * 8, CE * 8)],
                    oute_hbm.at[pl.ds(0, CE * 8)], souts[b]).wait()

        def make_edge_group(rij_v, et_v, oute_v):
            def edge_group(s):
                r = rij_v[pl.ds(s, L)]
                t = et_v[pl.ds(s, L)]
                rf = (r - d0) * inv_dx
                i0 = rf.astype(jnp.int32)
                i0 = jnp.clip(i0, 0, G - 2)
                w = rf - i0.astype(jnp.float32)
                base = (t * G + i0) * R
                ob = (s // LANE) * (8 * LANE) + lax.rem(s, LANE)
                hi = jnp.full((L,), -65536, jnp.int32)
                for rr in range(R):
                    yp = plsc.load_gather(tab_v, [base + rr])
                    y0 = plsc.bitcast(yp & hi, jnp.float32)
                    dy = plsc.bitcast(yp << 16, jnp.float32)
                    o = y0 + w * dy
                    rt, rs = rr // 8, rr % 8
                    oute_v[pl.ds(ob + (rt * CE * 8 + rs * LANE), L)] = o
            return edge_group

        groups = tuple(make_edge_group(rij_vs[b], et_vs[b], oute_vs[b])
                       for b in range(2))

        start_in(lo, 0)
        start_in(lo + 1, 1)

        def edge_pair(k, _):
            for b in range(2):
                crel = 2 * k + b

                @pl.when(crel < cnt)
                def _():
                    wait_in(b)

                    @pl.when(crel >= 2)
                    def _():
                        wait_out(b)

                    plsc.parallel_loop(0, CE, step=L, unroll=4)(groups[b])
                    start_out(lo + crel, b)

                    @pl.when(crel + 2 < cnt)
                    def _():
                        start_in(lo + crel + 2, b)
            return 0

        lax.fori_loop(0, (ncw + 2) // 2, edge_pair, 0)

        abase = wid * CA
        pltpu.sync_copy(at_hbm.at[pl.ds(abase, CA)], at_v)

        @plsc.parallel_loop(0, CA, step=L, unroll=4)
        def atom_group(s):
            at = at_v[pl.ds(s, L)]
            ab = at * NO
            ob = (s // LANE) * (NO * LANE) + lax.rem(s, LANE)
            for oo in range(NO):
                v = plsc.load_gather(ons_v, [ab + oo])
                outn_v[pl.ds(ob + oo * LANE, L)] = v

        pltpu.sync_copy(outn_v, outn_hbm.at[pl.ds(abase * NO, CA * NO)])

        wait_out(0)
        wait_out(1)

    return sc_kernel, NPAD, RT


def kernel(rij, edge_type, atom_type, distance_param, hopping_param, onsite_param):
    E = rij.shape[0]
    N = atom_type.shape[0]
    T, R, G = hopping_param.shape
    NO = onsite_param.shape[1]

    info = plsc.get_sparse_core_info()
    NC, NS = info.num_cores, info.num_subcores

    sc_kernel, NPAD, RT = _build(E, N, T, R, G, NO, NC, NS)

    tgr = jnp.transpose(hopping_param, (0, 2, 1))
    y1f = jnp.concatenate([tgr[:, 1:], tgr[:, -1:]], axis=1)
    y0b = tgr.astype(jnp.bfloat16)
    dyb = (y1f - tgr).astype(jnp.bfloat16)
    hi16 = (jax.lax.bitcast_convert_type(y0b, jnp.uint16)).astype(jnp.uint32)
    lo16 = (jax.lax.bitcast_convert_type(dyb, jnp.uint16)).astype(jnp.uint32)
    tab = jax.lax.bitcast_convert_type(lo16 | (hi16 << 16), jnp.int32).reshape(-1)
    A = onsite_param.shape[0]
    ons = jnp.zeros((L,), jnp.float32).at[: A * NO].set(
        onsite_param[:, :, 0].reshape(-1).astype(jnp.float32))
    at_pad = jnp.zeros((NPAD,), jnp.int32).at[:N].set(atom_type.astype(jnp.int32))

    edge_flat, node_flat = sc_kernel(
        rij, edge_type.astype(jnp.int32), at_pad,
        distance_param, tab, ons)

    ef = edge_flat.reshape(RT, E // 128, 8, 128)
    ef = jnp.transpose(ef, (1, 3, 0, 2)).reshape(E, RT * 8)[:, :R]
    nf = node_flat.reshape(NPAD // 128, NO, 128)
    nf = jnp.transpose(nf, (0, 2, 1)).reshape(NPAD, NO)[:N]
    return ef, nf
